# baseline (speedup 1.0000x reference)
import functools

import jax
import jax.numpy as jnp
from jax import lax
from jax.experimental import pallas as pl
from jax.experimental.pallas import tpu as pltpu
from jax.experimental.pallas import tpu_sc as plsc

_B = 16
_MAX_L = 2048
_N = 16384
_E = 262144
_D = 128
_NC = 33
_LANES = 16
_NCORES = 2
_NSUB = 16
_NW = _NCORES * _NSUB


def _sc_gather_sum(h_V, e0, e1):
    per_w = _E // _NW
    C = 128
    n_chunks = per_w // C
    mesh = plsc.VectorSubcoreMesh(core_axis_name="c", subcore_axis_name="s")

    @functools.partial(
        pl.kernel,
        out_type=jax.ShapeDtypeStruct((_E, _D), jnp.float32),
        mesh=mesh,
        scratch_types=[
            pltpu.VMEM((2, C), jnp.int32),
            pltpu.VMEM((2, C), jnp.int32),
            pltpu.VMEM((2, C, _D), jnp.float32),
            pltpu.VMEM((2, C, _D), jnp.float32),
            pltpu.VMEM((2, C, _D), jnp.float32),
            pltpu.SemaphoreType.DMA((2,)),
            pltpu.SemaphoreType.DMA((2,)),
            pltpu.SemaphoreType.DMA((2,)),
        ],
    )
    def k(hv_hbm, e0_hbm, e1_hbm, out_hbm, i0s, i1s, a_s, b_s, g_s,
          isem, gsem, osem):
        wid = lax.axis_index("s") * _NCORES + lax.axis_index("c")
        base_w = wid * per_w

        def idx_start(ci, p):
            base = base_w + ci * C
            pltpu.async_copy(e0_hbm.at[pl.ds(base, C)], i0s.at[p],
                             isem.at[p])
            pltpu.async_copy(e1_hbm.at[pl.ds(base, C)], i1s.at[p],
                             isem.at[p])

        def idx_drain(p):
            pltpu.make_async_copy(
                e0_hbm.at[pl.ds(base_w, C)], i0s.at[p], isem.at[p]).wait()
            pltpu.make_async_copy(
                e1_hbm.at[pl.ds(base_w, C)], i1s.at[p], isem.at[p]).wait()

        def gather_start(p):
            pltpu.async_copy(hv_hbm.at[i0s.at[p]], a_s.at[p], gsem.at[p])
            pltpu.async_copy(hv_hbm.at[i1s.at[p]], b_s.at[p], gsem.at[p])

        def gather_drain(p):
            pltpu.make_async_copy(
                hv_hbm.at[i0s.at[p]], a_s.at[p], gsem.at[p]).wait()
            pltpu.make_async_copy(
                hv_hbm.at[i1s.at[p]], b_s.at[p], gsem.at[p]).wait()

        def store_drain(p):
            pltpu.make_async_copy(
                g_s.at[p], out_hbm.at[pl.ds(base_w, C)], osem.at[p]).wait()

        def compute_and_store(ci, p):
            def row(i, c2):
                for kk in range(_D // _LANES):
                    sl = pl.ds(kk * _LANES, _LANES)
                    g_s[p, i, sl] = a_s[p, i, sl] + b_s[p, i, sl]
                return c2

            lax.fori_loop(0, C, row, 0, unroll=4)
            base = base_w + ci * C
            pltpu.async_copy(g_s.at[p], out_hbm.at[pl.ds(base, C)],
                             osem.at[p])

        idx_start(0, 0)
        idx_drain(0)
        gather_start(0)
        idx_start(1, 1)

        def body(ci, carry):
            p = jnp.bitwise_and(ci, 1)
            q = 1 - p

            @pl.when(ci >= 2)
            def _():
                store_drain(p)

            idx_drain(p)
            gather_start(p)
            gather_drain(q)

            @pl.when(ci < n_chunks - 1)
            def _():
                idx_start(ci + 1, q)

            compute_and_store(ci - 1, q)
            return carry

        lax.fori_loop(1, n_chunks, body, 0)
        p_last = (n_chunks - 1) & 1
        gather_drain(p_last)
        compute_and_store(n_chunks - 1, p_last)
        store_drain(0)
        store_drain(1)

    return k(h_V, e0, e1)


def _sc_scatter_add(h_E2, e1, n_out_rows):
    NH = _N // _NCORES
    per_t = _E // _NSUB
    C = 128
    H = C // 128
    n_chunks = per_t // C
    rows_t = NH // _NSUB
    mesh = plsc.VectorSubcoreMesh(core_axis_name="c", subcore_axis_name="s")

    @functools.partial(
        pl.kernel,
        out_type=jax.ShapeDtypeStruct((n_out_rows, _D), jnp.float32),
        mesh=mesh,
        scratch_types=[
            pltpu.VMEM((2, H, 128), jnp.int32),
            pltpu.VMEM((2, H, 128), jnp.int32),
            pltpu.VMEM((2, C, _D), jnp.float32),
            pltpu.VMEM_SHARED((NH + 8, _D), jnp.float32),
            pltpu.SemaphoreType.DMA((2,)),
            pltpu.SemaphoreType.DMA((2,)),
            pltpu.SemaphoreType.DMA((2,)),
        ],
    )
    def k(he2_hbm, e1_hbm, agg_hbm, i_s, i2_s, x_s, acc_sh,
          isem, lsem, ssem):
        cid = lax.axis_index("c")
        sid = lax.axis_index("s")
        row_lo = cid * NH
        tile_row0 = sid * rows_t

        def zrow(i, c2):
            for kk in range(_D // _LANES):
                x_s[0, i, pl.ds(kk * _LANES, _LANES)] = jnp.zeros(
                    (_LANES,), jnp.float32)
            return c2

        lax.fori_loop(0, C, zrow, 0)
        for r in range(rows_t // C):
            pltpu.sync_copy(x_s.at[0], acc_sh.at[pl.ds(tile_row0 + r * C, C)])
        plsc.subcore_barrier()

        def load_start(ci, p):
            base = sid * per_t + ci * C
            for h in range(H):
                pltpu.async_copy(e1_hbm.at[pl.ds(base + h * 128, 128)],
                                 i_s.at[p, h], isem.at[p])
            pltpu.async_copy(he2_hbm.at[pl.ds(base, C)], x_s.at[p],
                             lsem.at[p])

        def load_drain(p):
            for h in range(H):
                pltpu.make_async_copy(
                    e1_hbm.at[pl.ds(0, 128)], i_s.at[p, h],
                    isem.at[p]).wait()
            pltpu.make_async_copy(
                he2_hbm.at[pl.ds(0, C)], x_s.at[p], lsem.at[p]).wait()

        def scatter_drain(p):
            for h in range(H):
                pltpu.make_async_copy(
                    x_s.at[p, pl.ds(h * 128, 128)],
                    acc_sh.at[i2_s.at[p, h]], ssem.at[p]).wait()

        load_start(0, 0)

        def chunk(ci, carry):
            p = jnp.bitwise_and(ci, 1)
            q = 1 - p
            load_drain(p)
            for h in range(H):
                for kk in range(128 // _LANES):
                    sl = pl.ds(kk * _LANES, _LANES)
                    loc = i_s[p, h, sl] - row_lo
                    ok = (loc >= 0) & (loc < NH)
                    i2_s[p, h, sl] = jnp.where(ok, loc, NH)

            @pl.when(ci >= 1)
            def _():
                scatter_drain(q)

            @pl.when(ci < n_chunks - 1)
            def _():
                load_start(ci + 1, q)

            for h in range(H):
                pltpu.async_copy(x_s.at[p, pl.ds(h * 128, 128)],
                                 acc_sh.at[i2_s.at[p, h]], ssem.at[p],
                                 add=True)
            return carry

        lax.fori_loop(0, n_chunks, chunk, 0)
        scatter_drain((n_chunks - 1) & 1)
        plsc.subcore_barrier()

        for r in range(rows_t // C):
            pltpu.sync_copy(acc_sh.at[pl.ds(tile_row0 + r * C, C)], x_s.at[0])
            pltpu.sync_copy(
                x_s.at[0], agg_hbm.at[pl.ds(row_lo + tile_row0 + r * C, C)])

    return k(h_E2, e1)


def _tc_edge_mlp(h_E, G, W_edge):
    blk = 4096

    def body(he_ref, g_ref, w_ref, out_ref):
        x = he_ref[...] + g_ref[...]
        out_ref[...] = jnp.maximum(
            jnp.dot(x, w_ref[...], preferred_element_type=jnp.float32), 0.0)

    return pl.pallas_call(
        body,
        grid=(_E // blk,),
        in_specs=[
            pl.BlockSpec((blk, _D), lambda i: (i, 0)),
            pl.BlockSpec((blk, _D), lambda i: (i, 0)),
            pl.BlockSpec((_D, _D), lambda i: (0, 0)),
        ],
        out_specs=pl.BlockSpec((blk, _D), lambda i: (i, 0)),
        out_shape=jax.ShapeDtypeStruct((_E, _D), jnp.float32),
    )(h_E, G, W_edge)


def _tc_tail(hv_pad, agg_pad, W_embed, W_prob_pad, shift, nums):
    Lb = 1024
    nj = _MAX_L // Lb
    nsteps = _B * nj

    def body(shift_ref, nums_ref, hv_hbm, ag_hbm, we_ref, wp_ref,
             pred_ref, conf_ref, embeds_ref, probs_ref, mask_ref,
             hv_v, ag_v, sem0, sem1):
        b = pl.program_id(0)
        j = pl.program_id(1)
        t = b * nj + j
        p = lax.rem(t, 2)

        def start_for(tt, slot):
            bb = tt // nj
            jj = lax.rem(tt, nj)
            st = shift_ref[bb] + jj * Lb
            pltpu.make_async_copy(hv_hbm.at[pl.ds(st, Lb)], hv_v.at[slot],
                                  sem0.at[slot]).start()
            pltpu.make_async_copy(ag_hbm.at[pl.ds(st, Lb)], ag_v.at[slot],
                                  sem1.at[slot]).start()

        @pl.when(t == 0)
        def _():
            start_for(0, 0)

        @pl.when(t < nsteps - 1)
        def _():
            start_for(t + 1, 1 - p)

        pltpu.make_async_copy(hv_hbm.at[pl.ds(0, Lb)], hv_v.at[p],
                              sem0.at[p]).wait()
        pltpu.make_async_copy(ag_hbm.at[pl.ds(0, Lb)], ag_v.at[p],
                              sem1.at[p]).wait()

        x = hv_v[p] + ag_v[p]
        e = jnp.tanh(jnp.dot(x, we_ref[...],
                             preferred_element_type=jnp.float32))
        logits = jnp.dot(e, wp_ref[...], preferred_element_type=jnp.float32)
        col = lax.broadcasted_iota(jnp.int32, logits.shape, 1)
        lm = jnp.where(col < _NC, logits, -1e30)
        m = jnp.max(lm, axis=-1, keepdims=True)
        ex = jnp.exp(lm - m)
        probs = ex / jnp.sum(ex, axis=-1, keepdims=True)

        l = j * Lb + lax.broadcasted_iota(jnp.int32, (Lb, 1), 0)
        valid = l < nums_ref[b]
        conf = jnp.max(probs, axis=-1, keepdims=True)
        ids = lax.broadcasted_iota(jnp.int32, (Lb, 128), 1)
        pred = jnp.min(jnp.where(probs >= conf, ids, 128),
                       axis=-1, keepdims=True)
        conf_ref[...] = jnp.where(valid, conf, 1.0)
        pred_ref[...] = jnp.where(valid, pred, 1)
        mask_ref[...] = valid.astype(jnp.int32)
        embeds_ref[...] = jnp.where(valid[None, :, :], e[None], 1.0)
        probs_ref[...] = jnp.where(valid[None, :, :], probs[None, :, :_NC],
                                   1.0)

    grid_spec = pltpu.PrefetchScalarGridSpec(
        num_scalar_prefetch=2,
        grid=(_B, nj),
        in_specs=[
            pl.BlockSpec(memory_space=pl.ANY),
            pl.BlockSpec(memory_space=pl.ANY),
            pl.BlockSpec((_D, _D), lambda b, j, *_: (0, 0)),
            pl.BlockSpec((_D, 128), lambda b, j, *_: (0, 0)),
        ],
        out_specs=[
            pl.BlockSpec((Lb, 1), lambda b, j, *_: (b * nj + j, 0)),
            pl.BlockSpec((Lb, 1), lambda b, j, *_: (b * nj + j, 0)),
            pl.BlockSpec((1, Lb, _D), lambda b, j, *_: (b, j, 0)),
            pl.BlockSpec((1, Lb, _NC), lambda b, j, *_: (b, j, 0)),
            pl.BlockSpec((Lb, 1), lambda b, j, *_: (b * nj + j, 0)),
        ],
        scratch_shapes=[
            pltpu.VMEM((2, Lb, _D), jnp.float32),
            pltpu.VMEM((2, Lb, _D), jnp.float32),
            pltpu.SemaphoreType.DMA((2,)),
            pltpu.SemaphoreType.DMA((2,)),
        ],
    )
    pred_f, conf_f, embeds, probs_out, mask_f = pl.pallas_call(
        body,
        grid_spec=grid_spec,
        out_shape=[
            jax.ShapeDtypeStruct((_B * _MAX_L, 1), jnp.int32),
            jax.ShapeDtypeStruct((_B * _MAX_L, 1), jnp.float32),
            jax.ShapeDtypeStruct((_B, _MAX_L, _D), jnp.float32),
            jax.ShapeDtypeStruct((_B, _MAX_L, _NC), jnp.float32),
            jax.ShapeDtypeStruct((_B * _MAX_L, 1), jnp.int32),
        ],
    )(shift, nums, hv_pad, agg_pad, W_embed, W_prob_pad)
    pred_ids = pred_f.reshape(_B, _MAX_L)
    confs = conf_f.reshape(_B, _MAX_L)
    attention_mask = mask_f.reshape(_B, _MAX_L).astype(bool)
    return pred_ids, confs, embeds, probs_out, attention_mask


def kernel(h_V, h_E, E_idx, batch_id, W_edge, W_embed, W_prob):
    e0 = E_idx[0]
    e1 = E_idx[1]
    nums = jnp.sum(
        batch_id[None, :] == jnp.arange(_B, dtype=jnp.int32)[:, None],
        axis=1, dtype=jnp.int32)
    shift = jnp.concatenate(
        [jnp.zeros((1,), jnp.int32), jnp.cumsum(nums, dtype=jnp.int32)])

    G = _sc_gather_sum(h_V, e0, e1)
    h_E2 = _tc_edge_mlp(h_E, G, W_edge)
    agg_pad = _sc_scatter_add(h_E2, e1, _N + _MAX_L)
    hv_pad = jnp.pad(h_V, ((0, _MAX_L), (0, 0)))
    W_prob_pad = jnp.pad(W_prob, ((0, 0), (0, 128 - _NC)))
    pred_ids, confs, embeds, probs_out, attention_mask = _tc_tail(
        hv_pad, agg_pad, W_embed, W_prob_pad, shift, nums)
    return pred_ids, confs, embeds, probs_out, attention_mask, h_E2


# speedup vs baseline: 1.1523x; 1.1523x over previous
import functools

import jax
import jax.numpy as jnp
from jax import lax
from jax.experimental import pallas as pl
from jax.experimental.pallas import tpu as pltpu
from jax.experimental.pallas import tpu_sc as plsc

_B = 16
_MAX_L = 2048
_N = 16384
_E = 262144
_D = 128
_NC = 33
_LANES = 16
_NCORES = 2
_NSUB = 16
_NW = _NCORES * _NSUB


def _sc_gather_sum(h_V, e0, e1):
    per_w = _E // _NW
    C = 128
    n_chunks = per_w // C
    mesh = plsc.VectorSubcoreMesh(core_axis_name="c", subcore_axis_name="s")

    @functools.partial(
        pl.kernel,
        out_type=jax.ShapeDtypeStruct((_E, _D), jnp.float32),
        mesh=mesh,
        scratch_types=[
            pltpu.VMEM((2, C), jnp.int32),
            pltpu.VMEM((2, C), jnp.int32),
            pltpu.VMEM((2, C, _D), jnp.float32),
            pltpu.VMEM((2, C, _D), jnp.float32),
            pltpu.VMEM((2, C, _D), jnp.float32),
            pltpu.SemaphoreType.DMA((2,)),
            pltpu.SemaphoreType.DMA((2,)),
            pltpu.SemaphoreType.DMA((2,)),
        ],
    )
    def k(hv_hbm, e0_hbm, e1_hbm, out_hbm, i0s, i1s, a_s, b_s, g_s,
          isem, gsem, osem):
        wid = lax.axis_index("s") * _NCORES + lax.axis_index("c")
        base_w = wid * per_w

        def idx_start(ci, p):
            base = base_w + ci * C
            pltpu.async_copy(e0_hbm.at[pl.ds(base, C)], i0s.at[p],
                             isem.at[p])
            pltpu.async_copy(e1_hbm.at[pl.ds(base, C)], i1s.at[p],
                             isem.at[p])

        def idx_drain(p):
            pltpu.make_async_copy(
                e0_hbm.at[pl.ds(base_w, C)], i0s.at[p], isem.at[p]).wait()
            pltpu.make_async_copy(
                e1_hbm.at[pl.ds(base_w, C)], i1s.at[p], isem.at[p]).wait()

        def gather_start(p):
            pltpu.async_copy(hv_hbm.at[i0s.at[p]], a_s.at[p], gsem.at[p])
            pltpu.async_copy(hv_hbm.at[i1s.at[p]], b_s.at[p], gsem.at[p])

        def gather_drain(p):
            pltpu.make_async_copy(
                hv_hbm.at[i0s.at[p]], a_s.at[p], gsem.at[p]).wait()
            pltpu.make_async_copy(
                hv_hbm.at[i1s.at[p]], b_s.at[p], gsem.at[p]).wait()

        def store_drain(p):
            pltpu.make_async_copy(
                g_s.at[p], out_hbm.at[pl.ds(base_w, C)], osem.at[p]).wait()

        def compute_and_store(ci, p):
            for pp in range(2):
                @pl.when(p == pp)
                def _():
                    def row(i, c2):
                        for kk in range(_D // _LANES):
                            sl = pl.ds(kk * _LANES, _LANES)
                            g_s[pp, i, sl] = a_s[pp, i, sl] + b_s[pp, i, sl]
                        return c2

                    lax.fori_loop(0, C, row, 0, unroll=4)

            base = base_w + ci * C
            pltpu.async_copy(g_s.at[p], out_hbm.at[pl.ds(base, C)],
                             osem.at[p])

        idx_start(0, 0)
        idx_drain(0)
        gather_start(0)
        idx_start(1, 1)

        def body(ci, carry):
            p = jnp.bitwise_and(ci, 1)
            q = 1 - p

            @pl.when(ci >= 2)
            def _():
                store_drain(p)

            idx_drain(p)
            gather_start(p)
            gather_drain(q)

            @pl.when(ci < n_chunks - 1)
            def _():
                idx_start(ci + 1, q)

            compute_and_store(ci - 1, q)
            return carry

        lax.fori_loop(1, n_chunks, body, 0)
        p_last = (n_chunks - 1) & 1
        gather_drain(p_last)
        compute_and_store(n_chunks - 1, p_last)
        store_drain(0)
        store_drain(1)

    return k(h_V, e0, e1)


def _sc_scatter_add(h_E2, e1, n_out_rows):
    NH = _N // _NCORES
    per_t = _E // _NSUB
    C = 128
    H = C // 128
    n_chunks = per_t // C
    rows_t = NH // _NSUB
    mesh = plsc.VectorSubcoreMesh(core_axis_name="c", subcore_axis_name="s")

    @functools.partial(
        pl.kernel,
        out_type=jax.ShapeDtypeStruct((n_out_rows, _D), jnp.float32),
        mesh=mesh,
        scratch_types=[
            pltpu.VMEM((2, H, 128), jnp.int32),
            pltpu.VMEM((2, H, 128), jnp.int32),
            pltpu.VMEM((2, C, _D), jnp.float32),
            pltpu.VMEM_SHARED((NH + 8, _D), jnp.float32),
            pltpu.SemaphoreType.DMA((2,)),
            pltpu.SemaphoreType.DMA((2,)),
            pltpu.SemaphoreType.DMA((2,)),
        ],
    )
    def k(he2_hbm, e1_hbm, agg_hbm, i_s, i2_s, x_s, acc_sh,
          isem, lsem, ssem):
        cid = lax.axis_index("c")
        sid = lax.axis_index("s")
        row_lo = cid * NH
        tile_row0 = sid * rows_t

        def zrow(i, c2):
            for kk in range(_D // _LANES):
                x_s[0, i, pl.ds(kk * _LANES, _LANES)] = jnp.zeros(
                    (_LANES,), jnp.float32)
            return c2

        lax.fori_loop(0, C, zrow, 0)
        for r in range(rows_t // C):
            pltpu.sync_copy(x_s.at[0], acc_sh.at[pl.ds(tile_row0 + r * C, C)])
        plsc.subcore_barrier()

        def load_start(ci, p):
            base = sid * per_t + ci * C
            for h in range(H):
                pltpu.async_copy(e1_hbm.at[pl.ds(base + h * 128, 128)],
                                 i_s.at[p, h], isem.at[p])
            pltpu.async_copy(he2_hbm.at[pl.ds(base, C)], x_s.at[p],
                             lsem.at[p])

        def load_drain(p):
            for h in range(H):
                pltpu.make_async_copy(
                    e1_hbm.at[pl.ds(0, 128)], i_s.at[p, h],
                    isem.at[p]).wait()
            pltpu.make_async_copy(
                he2_hbm.at[pl.ds(0, C)], x_s.at[p], lsem.at[p]).wait()

        def scatter_drain(p):
            for h in range(H):
                pltpu.make_async_copy(
                    x_s.at[p, pl.ds(h * 128, 128)],
                    acc_sh.at[i2_s.at[p, h]], ssem.at[p]).wait()

        load_start(0, 0)

        def chunk(ci, carry):
            p = jnp.bitwise_and(ci, 1)
            q = 1 - p
            load_drain(p)
            for h in range(H):
                for kk in range(128 // _LANES):
                    sl = pl.ds(kk * _LANES, _LANES)
                    loc = i_s[p, h, sl] - row_lo
                    ok = (loc >= 0) & (loc < NH)
                    i2_s[p, h, sl] = jnp.where(ok, loc, NH)

            @pl.when(ci >= 1)
            def _():
                scatter_drain(q)

            @pl.when(ci < n_chunks - 1)
            def _():
                load_start(ci + 1, q)

            for h in range(H):
                pltpu.async_copy(x_s.at[p, pl.ds(h * 128, 128)],
                                 acc_sh.at[i2_s.at[p, h]], ssem.at[p],
                                 add=True)
            return carry

        lax.fori_loop(0, n_chunks, chunk, 0)
        scatter_drain((n_chunks - 1) & 1)
        plsc.subcore_barrier()

        for r in range(rows_t // C):
            pltpu.sync_copy(acc_sh.at[pl.ds(tile_row0 + r * C, C)], x_s.at[0])
            pltpu.sync_copy(
                x_s.at[0], agg_hbm.at[pl.ds(row_lo + tile_row0 + r * C, C)])

    return k(h_E2, e1)


def _tc_edge_mlp(h_E, G, W_edge):
    blk = 4096

    def body(he_ref, g_ref, w_ref, out_ref):
        x = he_ref[...] + g_ref[...]
        out_ref[...] = jnp.maximum(
            jnp.dot(x, w_ref[...], preferred_element_type=jnp.float32), 0.0)

    return pl.pallas_call(
        body,
        grid=(_E // blk,),
        in_specs=[
            pl.BlockSpec((blk, _D), lambda i: (i, 0)),
            pl.BlockSpec((blk, _D), lambda i: (i, 0)),
            pl.BlockSpec((_D, _D), lambda i: (0, 0)),
        ],
        out_specs=pl.BlockSpec((blk, _D), lambda i: (i, 0)),
        out_shape=jax.ShapeDtypeStruct((_E, _D), jnp.float32),
    )(h_E, G, W_edge)


def _tc_tail(hv_pad, agg_pad, W_embed, W_prob_pad, shift, nums):
    Lb = 1024
    nj = _MAX_L // Lb
    nsteps = _B * nj

    def body(shift_ref, nums_ref, hv_hbm, ag_hbm, we_ref, wp_ref,
             pred_ref, conf_ref, embeds_ref, probs_ref, mask_ref,
             hv_v, ag_v, sem0, sem1):
        b = pl.program_id(0)
        j = pl.program_id(1)
        t = b * nj + j
        p = lax.rem(t, 2)

        def start_for(tt, slot):
            bb = tt // nj
            jj = lax.rem(tt, nj)
            st = shift_ref[bb] + jj * Lb
            pltpu.make_async_copy(hv_hbm.at[pl.ds(st, Lb)], hv_v.at[slot],
                                  sem0.at[slot]).start()
            pltpu.make_async_copy(ag_hbm.at[pl.ds(st, Lb)], ag_v.at[slot],
                                  sem1.at[slot]).start()

        @pl.when(t == 0)
        def _():
            start_for(0, 0)

        @pl.when(t < nsteps - 1)
        def _():
            start_for(t + 1, 1 - p)

        pltpu.make_async_copy(hv_hbm.at[pl.ds(0, Lb)], hv_v.at[p],
                              sem0.at[p]).wait()
        pltpu.make_async_copy(ag_hbm.at[pl.ds(0, Lb)], ag_v.at[p],
                              sem1.at[p]).wait()

        x = hv_v[p] + ag_v[p]
        e = jnp.tanh(jnp.dot(x, we_ref[...],
                             preferred_element_type=jnp.float32))
        logits = jnp.dot(e, wp_ref[...], preferred_element_type=jnp.float32)
        col = lax.broadcasted_iota(jnp.int32, logits.shape, 1)
        lm = jnp.where(col < _NC, logits, -1e30)
        m = jnp.max(lm, axis=-1, keepdims=True)
        ex = jnp.exp(lm - m)
        probs = ex / jnp.sum(ex, axis=-1, keepdims=True)

        l = j * Lb + lax.broadcasted_iota(jnp.int32, (Lb, 1), 0)
        valid = l < nums_ref[b]
        conf = jnp.max(probs, axis=-1, keepdims=True)
        ids = lax.broadcasted_iota(jnp.int32, (Lb, 128), 1)
        pred = jnp.min(jnp.where(probs >= conf, ids, 128),
                       axis=-1, keepdims=True)
        conf_ref[...] = jnp.where(valid, conf, 1.0)
        pred_ref[...] = jnp.where(valid, pred, 1)
        mask_ref[...] = valid.astype(jnp.int32)
        embeds_ref[...] = jnp.where(valid[None, :, :], e[None], 1.0)
        probs_ref[...] = jnp.where(valid[None, :, :], probs[None, :, :_NC],
                                   1.0)

    grid_spec = pltpu.PrefetchScalarGridSpec(
        num_scalar_prefetch=2,
        grid=(_B, nj),
        in_specs=[
            pl.BlockSpec(memory_space=pl.ANY),
            pl.BlockSpec(memory_space=pl.ANY),
            pl.BlockSpec((_D, _D), lambda b, j, *_: (0, 0)),
            pl.BlockSpec((_D, 128), lambda b, j, *_: (0, 0)),
        ],
        out_specs=[
            pl.BlockSpec((Lb, 1), lambda b, j, *_: (b * nj + j, 0)),
            pl.BlockSpec((Lb, 1), lambda b, j, *_: (b * nj + j, 0)),
            pl.BlockSpec((1, Lb, _D), lambda b, j, *_: (b, j, 0)),
            pl.BlockSpec((1, Lb, _NC), lambda b, j, *_: (b, j, 0)),
            pl.BlockSpec((Lb, 1), lambda b, j, *_: (b * nj + j, 0)),
        ],
        scratch_shapes=[
            pltpu.VMEM((2, Lb, _D), jnp.float32),
            pltpu.VMEM((2, Lb, _D), jnp.float32),
            pltpu.SemaphoreType.DMA((2,)),
            pltpu.SemaphoreType.DMA((2,)),
        ],
    )
    pred_f, conf_f, embeds, probs_out, mask_f = pl.pallas_call(
        body,
        grid_spec=grid_spec,
        out_shape=[
            jax.ShapeDtypeStruct((_B * _MAX_L, 1), jnp.int32),
            jax.ShapeDtypeStruct((_B * _MAX_L, 1), jnp.float32),
            jax.ShapeDtypeStruct((_B, _MAX_L, _D), jnp.float32),
            jax.ShapeDtypeStruct((_B, _MAX_L, _NC), jnp.float32),
            jax.ShapeDtypeStruct((_B * _MAX_L, 1), jnp.int32),
        ],
    )(shift, nums, hv_pad, agg_pad, W_embed, W_prob_pad)
    pred_ids = pred_f.reshape(_B, _MAX_L)
    confs = conf_f.reshape(_B, _MAX_L)
    attention_mask = mask_f.reshape(_B, _MAX_L).astype(bool)
    return pred_ids, confs, embeds, probs_out, attention_mask


def kernel(h_V, h_E, E_idx, batch_id, W_edge, W_embed, W_prob):
    e0 = E_idx[0]
    e1 = E_idx[1]
    nums = jnp.sum(
        batch_id[None, :] == jnp.arange(_B, dtype=jnp.int32)[:, None],
        axis=1, dtype=jnp.int32)
    shift = jnp.concatenate(
        [jnp.zeros((1,), jnp.int32), jnp.cumsum(nums, dtype=jnp.int32)])

    G = _sc_gather_sum(h_V, e0, e1)
    h_E2 = _tc_edge_mlp(h_E, G, W_edge)
    agg_pad = _sc_scatter_add(h_E2, e1, _N + _MAX_L)
    hv_pad = jnp.pad(h_V, ((0, _MAX_L), (0, 0)))
    W_prob_pad = jnp.pad(W_prob, ((0, 0), (0, 128 - _NC)))
    pred_ids, confs, embeds, probs_out, attention_mask = _tc_tail(
        hv_pad, agg_pad, W_embed, W_prob_pad, shift, nums)
    return pred_ids, confs, embeds, probs_out, attention_mask, h_E2
